# Initial kernel scaffold; baseline (speedup 1.0000x reference)
import jax
import jax.numpy as jnp
from jax.experimental import pallas as pl


def kernel(x, feat_w, W1, b1, W2, b2):
    raise NotImplementedError("write your pallas kernel here")


# speedup vs baseline: 1.1152x; 1.1152x over previous
import functools

import jax
import jax.numpy as jnp
from jax.experimental import pallas as pl
from jax.experimental.pallas import tpu as pltpu

_EPS = 1e-12


def _prep_body(fw_ref, w1_ref, mn_ref, vp_ref, *, c):
    fw = fw_ref[...]
    nf = jnp.sqrt(jnp.sum(fw * fw, axis=1, keepdims=True))
    mn_full = fw / jnp.maximum(nf, _EPS)
    vp_ref[...] = jnp.dot(mn_full, w1_ref[...],
                          preferred_element_type=jnp.float32)
    m = fw[:, :c]
    nm = jnp.sqrt(jnp.sum(m * m, axis=1, keepdims=True))
    mn_ref[...] = m / jnp.maximum(nm, _EPS)


def _flash_body(x_ref, mn_ref, vp_ref, b1_ref, w2_ref, b2_ref, o_ref,
                q_scr, acc_scr, ssum_scr):
    j = pl.program_id(1)

    @pl.when(j == 0)
    def _init():
        xq = x_ref[...]
        nq = jnp.sqrt(jnp.sum(xq * xq, axis=1, keepdims=True))
        q_scr[...] = xq / jnp.maximum(nq, _EPS)
        acc_scr[...] = jnp.zeros_like(acc_scr)
        ssum_scr[...] = jnp.zeros_like(ssum_scr)

    s = jax.lax.dot_general(q_scr[...], mn_ref[...],
                            (((1,), (1,)), ((), ())),
                            preferred_element_type=jnp.float32)
    p = jnp.exp(s)
    ssum_scr[...] += jnp.sum(p, axis=1, keepdims=True)
    acc_scr[...] += jnp.dot(p, vp_ref[...],
                            preferred_element_type=jnp.float32)

    @pl.when(j == pl.num_programs(1) - 1)
    def _fin():
        z = acc_scr[...] / ssum_scr[...] + b1_ref[...]
        h1 = 0.5 * z * (1.0 + jax.lax.erf(z * (2.0 ** -0.5)))
        o_ref[...] = jnp.dot(h1, w2_ref[...],
                             preferred_element_type=jnp.float32) + b2_ref[...]


def kernel(x, feat_w, W1, b1, W2, b2):
    b, c, h, w = x.shape
    n = b * h * w
    kdim, cf = feat_w.shape
    hdim = W1.shape[1]
    x_flat = jnp.transpose(x, (0, 2, 3, 1)).reshape(n, c)

    BKP = 1024
    mn, vp = pl.pallas_call(
        functools.partial(_prep_body, c=c),
        grid=(kdim // BKP,),
        in_specs=[pl.BlockSpec((BKP, cf), lambda i: (i, 0)),
                  pl.BlockSpec((cf, hdim), lambda i: (0, 0))],
        out_specs=[pl.BlockSpec((BKP, c), lambda i: (i, 0)),
                   pl.BlockSpec((BKP, hdim), lambda i: (i, 0))],
        out_shape=[jax.ShapeDtypeStruct((kdim, c), jnp.float32),
                   jax.ShapeDtypeStruct((kdim, hdim), jnp.float32)],
    )(feat_w, W1)

    BQ, BK = 512, 512
    out = pl.pallas_call(
        _flash_body,
        grid=(n // BQ, kdim // BK),
        in_specs=[pl.BlockSpec((BQ, c), lambda i, j: (i, 0)),
                  pl.BlockSpec((BK, c), lambda i, j: (j, 0)),
                  pl.BlockSpec((BK, hdim), lambda i, j: (j, 0)),
                  pl.BlockSpec((1, hdim), lambda i, j: (0, 0)),
                  pl.BlockSpec((hdim, hdim), lambda i, j: (0, 0)),
                  pl.BlockSpec((1, hdim), lambda i, j: (0, 0))],
        out_specs=pl.BlockSpec((BQ, hdim), lambda i, j: (i, 0)),
        out_shape=jax.ShapeDtypeStruct((n, hdim), jnp.float32),
        scratch_shapes=[pltpu.VMEM((BQ, c), jnp.float32),
                        pltpu.VMEM((BQ, hdim), jnp.float32),
                        pltpu.VMEM((BQ, 1), jnp.float32)],
        compiler_params=pltpu.CompilerParams(
            dimension_semantics=("parallel", "arbitrary")),
    )(x_flat, mn, vp, b1.reshape(1, hdim), W2, b2.reshape(1, hdim))

    return jnp.transpose(out.reshape(b, h, w, hdim), (0, 3, 1, 2))


# speedup vs baseline: 1.1960x; 1.0724x over previous
import functools

import jax
import jax.numpy as jnp
from jax.experimental import pallas as pl
from jax.experimental.pallas import tpu as pltpu

_EPS = 1e-12


def _prep_body(fw_ref, w1_ref, mn_ref, vp_ref, *, c):
    fw = fw_ref[...]
    nf = jnp.sqrt(jnp.sum(fw * fw, axis=1, keepdims=True))
    mn_full = fw / jnp.maximum(nf, _EPS)
    vp_ref[...] = jnp.dot(mn_full, w1_ref[...],
                          preferred_element_type=jnp.float32
                          ).astype(jnp.bfloat16)
    m = fw[:, :c]
    nm = jnp.sqrt(jnp.sum(m * m, axis=1, keepdims=True))
    mn_ref[...] = (m / jnp.maximum(nm, _EPS)).astype(jnp.bfloat16)


def _flash_body(x_ref, mn_ref, vp_ref, b1_ref, w2_ref, b2_ref, o_ref,
                q_scr, acc_scr, ssum_scr):
    j = pl.program_id(1)

    @pl.when(j == 0)
    def _init():
        xq = x_ref[...]
        nq = jnp.sqrt(jnp.sum(xq * xq, axis=1, keepdims=True))
        q_scr[...] = (xq / jnp.maximum(nq, _EPS)).astype(jnp.bfloat16)
        acc_scr[...] = jnp.zeros_like(acc_scr)
        ssum_scr[...] = jnp.zeros_like(ssum_scr)

    s = jax.lax.dot_general(q_scr[...], mn_ref[...],
                            (((1,), (1,)), ((), ())),
                            preferred_element_type=jnp.float32)
    p = jnp.exp(s)
    ssum_scr[...] += jnp.sum(p, axis=1, keepdims=True)
    acc_scr[...] += jnp.dot(p.astype(jnp.bfloat16), vp_ref[...],
                            preferred_element_type=jnp.float32)

    @pl.when(j == pl.num_programs(1) - 1)
    def _fin():
        z = acc_scr[...] / ssum_scr[...] + b1_ref[...]
        h1 = 0.5 * z * (1.0 + jax.lax.erf(z * (2.0 ** -0.5)))
        o_ref[...] = jnp.dot(h1.astype(jnp.bfloat16), w2_ref[...],
                             preferred_element_type=jnp.float32) + b2_ref[...]


def kernel(x, feat_w, W1, b1, W2, b2):
    b, c, h, w = x.shape
    n = b * h * w
    kdim, cf = feat_w.shape
    hdim = W1.shape[1]
    x_flat = jnp.transpose(x, (0, 2, 3, 1)).reshape(n, c)

    BKP = 1024
    mn, vp = pl.pallas_call(
        functools.partial(_prep_body, c=c),
        grid=(kdim // BKP,),
        in_specs=[pl.BlockSpec((BKP, cf), lambda i: (i, 0)),
                  pl.BlockSpec((cf, hdim), lambda i: (0, 0))],
        out_specs=[pl.BlockSpec((BKP, c), lambda i: (i, 0)),
                   pl.BlockSpec((BKP, hdim), lambda i: (i, 0))],
        out_shape=[jax.ShapeDtypeStruct((kdim, c), jnp.bfloat16),
                   jax.ShapeDtypeStruct((kdim, hdim), jnp.bfloat16)],
    )(feat_w, W1)

    BQ, BK = 512, 512
    out = pl.pallas_call(
        _flash_body,
        grid=(n // BQ, kdim // BK),
        in_specs=[pl.BlockSpec((BQ, c), lambda i, j: (i, 0)),
                  pl.BlockSpec((BK, c), lambda i, j: (j, 0)),
                  pl.BlockSpec((BK, hdim), lambda i, j: (j, 0)),
                  pl.BlockSpec((1, hdim), lambda i, j: (0, 0)),
                  pl.BlockSpec((hdim, hdim), lambda i, j: (0, 0)),
                  pl.BlockSpec((1, hdim), lambda i, j: (0, 0))],
        out_specs=pl.BlockSpec((BQ, hdim), lambda i, j: (i, 0)),
        out_shape=jax.ShapeDtypeStruct((n, hdim), jnp.float32),
        scratch_shapes=[pltpu.VMEM((BQ, c), jnp.bfloat16),
                        pltpu.VMEM((BQ, hdim), jnp.float32),
                        pltpu.VMEM((BQ, 1), jnp.float32)],
        compiler_params=pltpu.CompilerParams(
            dimension_semantics=("parallel", "arbitrary")),
    )(x_flat, mn, vp, b1.reshape(1, hdim), W2.astype(jnp.bfloat16),
      b2.reshape(1, hdim))

    return jnp.transpose(out.reshape(b, h, w, hdim), (0, 3, 1, 2))
